# baseline (speedup 1.0000x reference)
import functools

import jax
import jax.numpy as jnp
from jax import lax
from jax.experimental import pallas as pl
from jax.experimental.pallas import tpu as pltpu
from jax.experimental.pallas import tpu_sc as plsc

_SCORE_THRESHOLD = 0.05
_IOU_THRESHOLD = 0.5
_MAX_DET = 300
_NEG = float("-inf")

_B = 8
_N = 5000
_C = 80
_NPAD = 5008
_NCHUNK = _NPAD // 16
_CMPAD = 320
_OUTPAD = 304


def _prep_body(cls_ref, s_ref, lab_ref):
    x = cls_ref[...]
    xt = jnp.transpose(x, (2, 0, 1))
    best = xt[0]
    labv = jnp.zeros((_B, _N), jnp.int32)
    for c in range(1, _C):
        v = xt[c]
        upd = v > best
        best = jnp.where(upd, v, best)
        labv = jnp.where(upd, c, labv)
    s_ref[...] = jnp.where(best > _SCORE_THRESHOLD, best, _NEG)
    lab_ref[...] = labv


def _sc_nms(s_hbm, lab_hbm, box_hbm,
            oy1_hbm, ox1_hbm, oy2_hbm, ox2_hbm, osc_hbm, olab_hbm,
            s_v, lab_v, box_v, cm_v,
            ky1_v, kx1_v, ky2_v, kx2_v, osc_v, olab_v,
            sy1_v, sx1_v, sy2_v, sx2_v, sarea_v):
    w = lax.axis_index("s") * 2 + lax.axis_index("c")

    @pl.when(w < _B)
    def _():
        b = w
        pltpu.sync_copy(s_hbm.at[b], s_v.at[pl.ds(0, _N)])
        pltpu.sync_copy(lab_hbm.at[b], lab_v.at[pl.ds(0, _N)])
        pltpu.sync_copy(box_hbm.at[b], box_v.at[pl.ds(0, _N * 4)])

        iota = lax.iota(jnp.int32, 16)
        negv = jnp.full((16,), _NEG, jnp.float32)

        tail = s_v[pl.ds(_NPAD - 16, 16)]
        s_v[pl.ds(_NPAD - 16, 16)] = jnp.where(iota < _N - (_NPAD - 16),
                                               tail, negv)
        m1f = jnp.full((16,), -1.0, jnp.float32)
        m1i = jnp.full((16,), -1, jnp.int32)
        z16 = jnp.zeros((16,), jnp.float32)

        for k in range(_CMPAD // 16):
            cm_v[pl.ds(16 * k, 16)] = negv
        for k in range(_OUTPAD // 16):
            sl = pl.ds(16 * k, 16)
            ky1_v[sl] = m1f
            kx1_v[sl] = m1f
            ky2_v[sl] = m1f
            kx2_v[sl] = m1f
            osc_v[sl] = m1f
            olab_v[sl] = m1i
            sy1_v[sl] = m1f
            sx1_v[sl] = m1f
            sy2_v[sl] = m1f
            sx2_v[sl] = m1f
            sarea_v[sl] = z16

        def cmbody(k, _):
            m = jnp.max(s_v[pl.ds(k * 16, 16)])
            base = (k // 16) * 16
            lane = k - base
            old = cm_v[pl.ds(base, 16)]
            cm_v[pl.ds(base, 16)] = jnp.where(iota == lane,
                                              jnp.full((16,), m, jnp.float32),
                                              old)
            return 0
        lax.fori_loop(0, _NCHUNK, cmbody, 0)

        BIG = jnp.int32(10 ** 6)
        _NREG = _CMPAD // 16

        def _tree(op, vs):
            vs = list(vs)
            while len(vs) > 1:
                vs = [op(vs[i], vs[i + 1]) if i + 1 < len(vs) else vs[i]
                      for i in range(0, len(vs), 2)]
            return vs[0]

        cms0 = tuple(cm_v[pl.ds(16 * k, 16)] for k in range(_NREG))

        def cond(st):
            cnt, done = st[0], st[2]
            return jnp.logical_and(cnt < _MAX_DET, jnp.logical_not(done))

        def body(st):
            cnt, kcnt, done, cms = st[0], st[1], st[2], st[3]
            best = jnp.max(_tree(jnp.maximum, cms))
            valid = best > _NEG
            bestv = jnp.full((16,), best, jnp.float32)
            idxs = [jnp.where(cms[k] == bestv, iota + 16 * k, BIG)
                    for k in range(_NREG)]
            cstar = jnp.min(_tree(jnp.minimum, idxs))
            sv = s_v[pl.ds(cstar * 16, 16)]
            lanew = jnp.min(jnp.where(sv == bestv, iota, BIG))
            g = cstar * 16 + lanew
            gc = jnp.minimum(g, _N - 1)

            bv = box_v[pl.ds(gc * 4, 16)]
            cy1s = bv[0]
            cx1s = bv[1]
            cy2s = bv[2]
            cx2s = bv[3]
            clabs = lab_v[pl.ds(gc, 16)][0]
            cy1 = jnp.full((16,), cy1s, jnp.float32)
            cx1 = jnp.full((16,), cx1s, jnp.float32)
            cy2 = jnp.full((16,), cy2s, jnp.float32)
            cx2 = jnp.full((16,), cx2s, jnp.float32)
            careas = (cy2s - cy1s) * (cx2s - cx1s)
            carea = jnp.full((16,), careas, jnp.float32)

            newsv = jnp.where(iota == lanew, negv, sv)
            s_v[pl.ds(cstar * 16, 16)] = newsv
            newm = jnp.max(newsv)
            newmv = jnp.full((16,), newm, jnp.float32)
            kreg = cstar // 16
            klane = cstar - kreg * 16
            cms_new = tuple(
                jnp.where(kreg == k, jnp.where(iota == klane, newmv, cms[k]),
                          cms[k])
                for k in range(_NREG))

            def iou_supp(ky1, kx1, ky2, kx2, karea):
                yy1 = jnp.maximum(ky1, cy1)
                xx1 = jnp.maximum(kx1, cx1)
                yy2 = jnp.minimum(ky2, cy2)
                xx2 = jnp.minimum(kx2, cx2)
                inter = jnp.maximum(0.0, yy2 - yy1) * jnp.maximum(0.0, xx2 - xx1)
                union = karea + carea - inter
                iou = jnp.where(union > 0, inter / union, 0.0)
                return iou > _IOU_THRESHOLD

            cproper = jnp.logical_and(cy2s > cy1s, cx2s > cx1s)
            nk = jnp.where(cproper, (kcnt + 15) // 16, 0)

            def jbody(j, suppacc):
                sl = pl.ds(j * 16, 16)
                return jnp.logical_or(
                    suppacc,
                    iou_supp(sy1_v[sl], sx1_v[sl], sy2_v[sl], sx2_v[sl],
                             sarea_v[sl]))

            suppv = lax.fori_loop(0, nk, jbody, jnp.zeros((16,), jnp.bool_))
            anysupp = jnp.max(jnp.where(suppv, 1, 0).astype(jnp.int32)) > 0
            accept = jnp.logical_and(valid, jnp.logical_not(anysupp))

            obase = (cnt // 16) * 16
            olane = cnt - obase
            am = jnp.logical_and(iota == olane, jnp.full((16,), accept))
            osl = pl.ds(obase, 16)

            def blend_f(ref, vals):
                ref[osl] = jnp.where(am, jnp.full((16,), vals, jnp.float32),
                                     ref[osl])

            blend_f(ky1_v, cy1s)
            blend_f(kx1_v, cx1s)
            blend_f(ky2_v, cy2s)
            blend_f(kx2_v, cx2s)
            blend_f(osc_v, best)
            olab_v[osl] = jnp.where(am, jnp.full((16,), clabs - 1, jnp.int32),
                                    olab_v[osl])

            keep = jnp.logical_and(accept, cproper)
            kbase = (kcnt // 16) * 16
            klne = kcnt - kbase
            amk = jnp.logical_and(iota == klne, jnp.full((16,), keep))
            ksl = pl.ds(kbase, 16)

            def blend_k(ref, vals):
                ref[ksl] = jnp.where(amk, jnp.full((16,), vals, jnp.float32),
                                     ref[ksl])

            blend_k(sy1_v, cy1s)
            blend_k(sx1_v, cx1s)
            blend_k(sy2_v, cy2s)
            blend_k(sx2_v, cx2s)
            blend_k(sarea_v, careas)

            cnt2 = cnt + jnp.where(accept, 1, 0).astype(jnp.int32)
            kcnt2 = kcnt + jnp.where(keep, 1, 0).astype(jnp.int32)
            return cnt2, kcnt2, jnp.logical_not(valid), cms_new

        lax.while_loop(cond, body,
                       (jnp.int32(0), jnp.int32(0), jnp.bool_(False), cms0))

        pltpu.sync_copy(ky1_v, oy1_hbm.at[b])
        pltpu.sync_copy(kx1_v, ox1_hbm.at[b])
        pltpu.sync_copy(ky2_v, oy2_hbm.at[b])
        pltpu.sync_copy(kx2_v, ox2_hbm.at[b])
        pltpu.sync_copy(osc_v, osc_hbm.at[b])
        pltpu.sync_copy(olab_v, olab_hbm.at[b])


_sc_nms_call = functools.partial(
    pl.kernel,
    out_type=(
        jax.ShapeDtypeStruct((_B, _OUTPAD), jnp.float32),
        jax.ShapeDtypeStruct((_B, _OUTPAD), jnp.float32),
        jax.ShapeDtypeStruct((_B, _OUTPAD), jnp.float32),
        jax.ShapeDtypeStruct((_B, _OUTPAD), jnp.float32),
        jax.ShapeDtypeStruct((_B, _OUTPAD), jnp.float32),
        jax.ShapeDtypeStruct((_B, _OUTPAD), jnp.int32),
    ),
    mesh=plsc.VectorSubcoreMesh(core_axis_name="c", subcore_axis_name="s"),
    compiler_params=pltpu.CompilerParams(needs_layout_passes=False,
                                         use_tc_tiling_on_sc=False),
    scratch_types=[
        pltpu.VMEM((_NPAD,), jnp.float32),
        pltpu.VMEM((_NPAD + 16,), jnp.int32),
        pltpu.VMEM((_N * 4 + 16,), jnp.float32),
        pltpu.VMEM((_CMPAD,), jnp.float32),
        pltpu.VMEM((_OUTPAD,), jnp.float32),
        pltpu.VMEM((_OUTPAD,), jnp.float32),
        pltpu.VMEM((_OUTPAD,), jnp.float32),
        pltpu.VMEM((_OUTPAD,), jnp.float32),
        pltpu.VMEM((_OUTPAD,), jnp.float32),
        pltpu.VMEM((_OUTPAD,), jnp.int32),
        pltpu.VMEM((_OUTPAD,), jnp.float32),
        pltpu.VMEM((_OUTPAD,), jnp.float32),
        pltpu.VMEM((_OUTPAD,), jnp.float32),
        pltpu.VMEM((_OUTPAD,), jnp.float32),
        pltpu.VMEM((_OUTPAD,), jnp.float32),
    ],
)(_sc_nms)


def kernel(boxes, classification):
    s_pad, lab_pad = pl.pallas_call(
        _prep_body,
        out_shape=(
            jax.ShapeDtypeStruct((_B, _N), jnp.float32),
            jax.ShapeDtypeStruct((_B, _N), jnp.int32),
        ),
    )(classification)
    box_flat = boxes.reshape(_B, _N * 4)
    oy1, ox1, oy2, ox2, osc, olab = _sc_nms_call(s_pad, lab_pad, box_flat)
    out_boxes = jnp.stack(
        [oy1[:, :_MAX_DET], ox1[:, :_MAX_DET],
         oy2[:, :_MAX_DET], ox2[:, :_MAX_DET]], axis=-1)
    return out_boxes, osc[:, :_MAX_DET], olab[:, :_MAX_DET]


# speedup vs baseline: 1.1502x; 1.1502x over previous
import functools

import jax
import jax.numpy as jnp
from jax import lax
from jax.experimental import pallas as pl
from jax.experimental.pallas import tpu as pltpu
from jax.experimental.pallas import tpu_sc as plsc

_SCORE_THRESHOLD = 0.05
_IOU_THRESHOLD = 0.5
_MAX_DET = 300
_NEG = float("-inf")

_B = 8
_N = 5000
_C = 80
_NPAD = 5008
_NCHUNK = _NPAD // 16
_CMPAD = 320
_OUTPAD = 304


def _prep_body(cls_ref, s_ref, lab_ref):
    def cbody(c, carry):
        best, labv = carry
        v = cls_ref[c]
        upd = v > best
        return jnp.where(upd, v, best), jnp.where(upd, c, labv)

    best0 = cls_ref[0]
    lab0 = jnp.zeros((_B, _N), jnp.int32)
    best, labv = lax.fori_loop(1, _C, cbody, (best0, lab0))
    s_ref[...] = jnp.where(best > _SCORE_THRESHOLD, best, _NEG)
    lab_ref[...] = labv


def _sc_nms(s_hbm, lab_hbm, box_hbm,
            oy1_hbm, ox1_hbm, oy2_hbm, ox2_hbm, osc_hbm, olab_hbm,
            s_v, lab_v, box_v, cm_v,
            ky1_v, kx1_v, ky2_v, kx2_v, osc_v, olab_v,
            sy1_v, sx1_v, sy2_v, sx2_v, sarea_v):
    w = lax.axis_index("s") * 2 + lax.axis_index("c")

    @pl.when(w < _B)
    def _():
        b = w
        pltpu.sync_copy(s_hbm.at[b], s_v.at[pl.ds(0, _N)])
        pltpu.sync_copy(lab_hbm.at[b], lab_v.at[pl.ds(0, _N)])
        pltpu.sync_copy(box_hbm.at[b], box_v.at[pl.ds(0, _N * 4)])

        iota = lax.iota(jnp.int32, 16)
        negv = jnp.full((16,), _NEG, jnp.float32)

        tail = s_v[pl.ds(_NPAD - 16, 16)]
        s_v[pl.ds(_NPAD - 16, 16)] = jnp.where(iota < _N - (_NPAD - 16),
                                               tail, negv)
        m1f = jnp.full((16,), -1.0, jnp.float32)
        m1i = jnp.full((16,), -1, jnp.int32)
        z16 = jnp.zeros((16,), jnp.float32)

        for k in range(_CMPAD // 16):
            cm_v[pl.ds(16 * k, 16)] = negv
        for k in range(_OUTPAD // 16):
            sl = pl.ds(16 * k, 16)
            ky1_v[sl] = m1f
            kx1_v[sl] = m1f
            ky2_v[sl] = m1f
            kx2_v[sl] = m1f
            osc_v[sl] = m1f
            olab_v[sl] = m1i
            sy1_v[sl] = m1f
            sx1_v[sl] = m1f
            sy2_v[sl] = m1f
            sx2_v[sl] = m1f
            sarea_v[sl] = z16

        def cmbody(k, _):
            m = jnp.max(s_v[pl.ds(k * 16, 16)])
            base = (k // 16) * 16
            lane = k - base
            old = cm_v[pl.ds(base, 16)]
            cm_v[pl.ds(base, 16)] = jnp.where(iota == lane,
                                              jnp.full((16,), m, jnp.float32),
                                              old)
            return 0
        lax.fori_loop(0, _NCHUNK, cmbody, 0)

        BIG = jnp.int32(10 ** 6)
        _NREG = _CMPAD // 16

        def _tree(op, vs):
            vs = list(vs)
            while len(vs) > 1:
                vs = [op(vs[i], vs[i + 1]) if i + 1 < len(vs) else vs[i]
                      for i in range(0, len(vs), 2)]
            return vs[0]

        cms0 = tuple(cm_v[pl.ds(16 * k, 16)] for k in range(_NREG))

        def cond(st):
            cnt, done = st[0], st[2]
            return jnp.logical_and(cnt < _MAX_DET, jnp.logical_not(done))

        def body(st):
            cnt, kcnt, done, cms = st[0], st[1], st[2], st[3]
            best = jnp.max(_tree(jnp.maximum, cms))
            valid = best > _NEG
            bestv = jnp.full((16,), best, jnp.float32)
            idxs = [jnp.where(cms[k] == bestv, iota + 16 * k, BIG)
                    for k in range(_NREG)]
            cstar = jnp.min(_tree(jnp.minimum, idxs))
            sv = s_v[pl.ds(cstar * 16, 16)]
            lanew = jnp.min(jnp.where(sv == bestv, iota, BIG))
            g = cstar * 16 + lanew
            gc = jnp.minimum(g, _N - 1)

            bv = box_v[pl.ds(gc * 4, 16)]
            cy1s = bv[0]
            cx1s = bv[1]
            cy2s = bv[2]
            cx2s = bv[3]
            clabs = lab_v[pl.ds(gc, 16)][0]
            cy1 = jnp.full((16,), cy1s, jnp.float32)
            cx1 = jnp.full((16,), cx1s, jnp.float32)
            cy2 = jnp.full((16,), cy2s, jnp.float32)
            cx2 = jnp.full((16,), cx2s, jnp.float32)
            careas = (cy2s - cy1s) * (cx2s - cx1s)
            carea = jnp.full((16,), careas, jnp.float32)

            newsv = jnp.where(iota == lanew, negv, sv)
            s_v[pl.ds(cstar * 16, 16)] = newsv
            newm = jnp.max(newsv)
            newmv = jnp.full((16,), newm, jnp.float32)
            kreg = cstar // 16
            klane = cstar - kreg * 16
            cms_new = tuple(
                jnp.where(kreg == k, jnp.where(iota == klane, newmv, cms[k]),
                          cms[k])
                for k in range(_NREG))

            def iou_supp(ky1, kx1, ky2, kx2, karea):
                yy1 = jnp.maximum(ky1, cy1)
                xx1 = jnp.maximum(kx1, cx1)
                yy2 = jnp.minimum(ky2, cy2)
                xx2 = jnp.minimum(kx2, cx2)
                inter = jnp.maximum(0.0, yy2 - yy1) * jnp.maximum(0.0, xx2 - xx1)
                union = karea + carea - inter
                iou = jnp.where(union > 0, inter / union, 0.0)
                return iou > _IOU_THRESHOLD

            cproper = jnp.logical_and(cy2s > cy1s, cx2s > cx1s)
            nk = jnp.where(cproper, (kcnt + 15) // 16, 0)

            def jbody(j, suppacc):
                sl = pl.ds(j * 16, 16)
                return jnp.logical_or(
                    suppacc,
                    iou_supp(sy1_v[sl], sx1_v[sl], sy2_v[sl], sx2_v[sl],
                             sarea_v[sl]))

            suppv = lax.fori_loop(0, nk, jbody, jnp.zeros((16,), jnp.bool_))
            anysupp = jnp.max(jnp.where(suppv, 1, 0).astype(jnp.int32)) > 0
            accept = jnp.logical_and(valid, jnp.logical_not(anysupp))

            obase = (cnt // 16) * 16
            olane = cnt - obase
            am = jnp.logical_and(iota == olane, jnp.full((16,), accept))
            osl = pl.ds(obase, 16)

            def blend_f(ref, vals):
                ref[osl] = jnp.where(am, jnp.full((16,), vals, jnp.float32),
                                     ref[osl])

            blend_f(ky1_v, cy1s)
            blend_f(kx1_v, cx1s)
            blend_f(ky2_v, cy2s)
            blend_f(kx2_v, cx2s)
            blend_f(osc_v, best)
            olab_v[osl] = jnp.where(am, jnp.full((16,), clabs - 1, jnp.int32),
                                    olab_v[osl])

            keep = jnp.logical_and(accept, cproper)
            kbase = (kcnt // 16) * 16
            klne = kcnt - kbase
            amk = jnp.logical_and(iota == klne, jnp.full((16,), keep))
            ksl = pl.ds(kbase, 16)

            def blend_k(ref, vals):
                ref[ksl] = jnp.where(amk, jnp.full((16,), vals, jnp.float32),
                                     ref[ksl])

            blend_k(sy1_v, cy1s)
            blend_k(sx1_v, cx1s)
            blend_k(sy2_v, cy2s)
            blend_k(sx2_v, cx2s)
            blend_k(sarea_v, careas)

            cnt2 = cnt + jnp.where(accept, 1, 0).astype(jnp.int32)
            kcnt2 = kcnt + jnp.where(keep, 1, 0).astype(jnp.int32)
            return cnt2, kcnt2, jnp.logical_not(valid), cms_new

        lax.while_loop(cond, body,
                       (jnp.int32(0), jnp.int32(0), jnp.bool_(False), cms0))

        pltpu.sync_copy(ky1_v, oy1_hbm.at[b])
        pltpu.sync_copy(kx1_v, ox1_hbm.at[b])
        pltpu.sync_copy(ky2_v, oy2_hbm.at[b])
        pltpu.sync_copy(kx2_v, ox2_hbm.at[b])
        pltpu.sync_copy(osc_v, osc_hbm.at[b])
        pltpu.sync_copy(olab_v, olab_hbm.at[b])


_sc_nms_call = functools.partial(
    pl.kernel,
    out_type=(
        jax.ShapeDtypeStruct((_B, _OUTPAD), jnp.float32),
        jax.ShapeDtypeStruct((_B, _OUTPAD), jnp.float32),
        jax.ShapeDtypeStruct((_B, _OUTPAD), jnp.float32),
        jax.ShapeDtypeStruct((_B, _OUTPAD), jnp.float32),
        jax.ShapeDtypeStruct((_B, _OUTPAD), jnp.float32),
        jax.ShapeDtypeStruct((_B, _OUTPAD), jnp.int32),
    ),
    mesh=plsc.VectorSubcoreMesh(core_axis_name="c", subcore_axis_name="s"),
    compiler_params=pltpu.CompilerParams(needs_layout_passes=False,
                                         use_tc_tiling_on_sc=False),
    scratch_types=[
        pltpu.VMEM((_NPAD,), jnp.float32),
        pltpu.VMEM((_NPAD + 16,), jnp.int32),
        pltpu.VMEM((_N * 4 + 16,), jnp.float32),
        pltpu.VMEM((_CMPAD,), jnp.float32),
        pltpu.VMEM((_OUTPAD,), jnp.float32),
        pltpu.VMEM((_OUTPAD,), jnp.float32),
        pltpu.VMEM((_OUTPAD,), jnp.float32),
        pltpu.VMEM((_OUTPAD,), jnp.float32),
        pltpu.VMEM((_OUTPAD,), jnp.float32),
        pltpu.VMEM((_OUTPAD,), jnp.int32),
        pltpu.VMEM((_OUTPAD,), jnp.float32),
        pltpu.VMEM((_OUTPAD,), jnp.float32),
        pltpu.VMEM((_OUTPAD,), jnp.float32),
        pltpu.VMEM((_OUTPAD,), jnp.float32),
        pltpu.VMEM((_OUTPAD,), jnp.float32),
    ],
)(_sc_nms)


def kernel(boxes, classification):
    cls_t = jnp.transpose(classification, (2, 0, 1))
    s_pad, lab_pad = pl.pallas_call(
        _prep_body,
        out_shape=(
            jax.ShapeDtypeStruct((_B, _N), jnp.float32),
            jax.ShapeDtypeStruct((_B, _N), jnp.int32),
        ),
    )(cls_t)
    box_flat = boxes.reshape(_B, _N * 4)
    oy1, ox1, oy2, ox2, osc, olab = _sc_nms_call(s_pad, lab_pad, box_flat)
    out_boxes = jnp.stack(
        [oy1[:, :_MAX_DET], ox1[:, :_MAX_DET],
         oy2[:, :_MAX_DET], ox2[:, :_MAX_DET]], axis=-1)
    return out_boxes, osc[:, :_MAX_DET], olab[:, :_MAX_DET]


# speedup vs baseline: 1.1837x; 1.0292x over previous
import functools

import jax
import jax.numpy as jnp
from jax import lax
from jax.experimental import pallas as pl
from jax.experimental.pallas import tpu as pltpu
from jax.experimental.pallas import tpu_sc as plsc

_SCORE_THRESHOLD = 0.05
_IOU_THRESHOLD = 0.5
_MAX_DET = 300
_NEG = float("-inf")

_B = 8
_N = 5000
_C = 80
_NPAD = 5024
_NCHUNK = _NPAD // 32
_CMPAD = 160
_OUTPAD = 304


def _prep_body(cls_ref, s_ref, lab_ref):
    def cbody(c, carry):
        best, labv = carry
        v = cls_ref[c]
        upd = v > best
        return jnp.where(upd, v, best), jnp.where(upd, c, labv)

    best0 = cls_ref[0]
    lab0 = jnp.zeros((_B, _N), jnp.int32)
    best, labv = lax.fori_loop(1, _C, cbody, (best0, lab0))
    s_ref[...] = jnp.where(best > _SCORE_THRESHOLD, best, _NEG)
    lab_ref[...] = labv


def _sc_nms(s_hbm, lab_hbm, box_hbm,
            oy1_hbm, ox1_hbm, oy2_hbm, ox2_hbm, osc_hbm, olab_hbm,
            s_v, lab_v, box_v, cm_v,
            ky1_v, kx1_v, ky2_v, kx2_v, osc_v, olab_v,
            sy1_v, sx1_v, sy2_v, sx2_v, sarea_v):
    w = lax.axis_index("s") * 2 + lax.axis_index("c")

    @pl.when(w < _B)
    def _():
        b = w
        pltpu.sync_copy(s_hbm.at[b], s_v.at[pl.ds(0, _N)])
        pltpu.sync_copy(lab_hbm.at[b], lab_v.at[pl.ds(0, _N)])
        pltpu.sync_copy(box_hbm.at[b], box_v.at[pl.ds(0, _N * 4)])

        iota = lax.iota(jnp.int32, 16)
        negv = jnp.full((16,), _NEG, jnp.float32)

        tail = s_v[pl.ds(_NPAD - 32, 16)]
        s_v[pl.ds(_NPAD - 32, 16)] = jnp.where(iota < _N - (_NPAD - 32),
                                               tail, negv)
        s_v[pl.ds(_NPAD - 16, 16)] = negv
        m1f = jnp.full((16,), -1.0, jnp.float32)
        m1i = jnp.full((16,), -1, jnp.int32)
        z16 = jnp.zeros((16,), jnp.float32)

        for k in range(_CMPAD // 16):
            cm_v[pl.ds(16 * k, 16)] = negv
        for k in range(_OUTPAD // 16):
            sl = pl.ds(16 * k, 16)
            ky1_v[sl] = m1f
            kx1_v[sl] = m1f
            ky2_v[sl] = m1f
            kx2_v[sl] = m1f
            osc_v[sl] = m1f
            olab_v[sl] = m1i
            sy1_v[sl] = m1f
            sx1_v[sl] = m1f
            sy2_v[sl] = m1f
            sx2_v[sl] = m1f
            sarea_v[sl] = z16

        def cmbody(k, _):
            m = jnp.max(jnp.maximum(s_v[pl.ds(k * 32, 16)],
                                    s_v[pl.ds(k * 32 + 16, 16)]))
            base = (k // 16) * 16
            lane = k - base
            old = cm_v[pl.ds(base, 16)]
            cm_v[pl.ds(base, 16)] = jnp.where(iota == lane,
                                              jnp.full((16,), m, jnp.float32),
                                              old)
            return 0
        lax.fori_loop(0, _NCHUNK, cmbody, 0)

        BIG = jnp.int32(10 ** 6)
        _NREG = _CMPAD // 16

        def _tree(op, vs):
            vs = list(vs)
            while len(vs) > 1:
                vs = [op(vs[i], vs[i + 1]) if i + 1 < len(vs) else vs[i]
                      for i in range(0, len(vs), 2)]
            return vs[0]

        cms0 = tuple(cm_v[pl.ds(16 * k, 16)] for k in range(_NREG))

        def cond(st):
            cnt, done = st[0], st[2]
            return jnp.logical_and(cnt < _MAX_DET, jnp.logical_not(done))

        def body(st):
            cnt, kcnt, done, cms = st[0], st[1], st[2], st[3]
            best = jnp.max(_tree(jnp.maximum, cms))
            valid = best > _NEG
            bestv = jnp.full((16,), best, jnp.float32)
            idxs = [jnp.where(cms[k] == bestv, iota + 16 * k, BIG)
                    for k in range(_NREG)]
            cstar = jnp.min(_tree(jnp.minimum, idxs))
            sv0 = s_v[pl.ds(cstar * 32, 16)]
            sv1 = s_v[pl.ds(cstar * 32 + 16, 16)]
            l0 = jnp.min(jnp.where(sv0 == bestv, iota, BIG))
            l1 = jnp.min(jnp.where(sv1 == bestv, iota + 16, BIG))
            lanew = jnp.minimum(l0, l1)
            g = cstar * 32 + lanew
            gc = jnp.minimum(g, _N - 1)

            bv = box_v[pl.ds(gc * 4, 16)]
            cy1s = bv[0]
            cx1s = bv[1]
            cy2s = bv[2]
            cx2s = bv[3]
            clabs = lab_v[pl.ds(gc, 16)][0]
            cy1 = jnp.full((16,), cy1s, jnp.float32)
            cx1 = jnp.full((16,), cx1s, jnp.float32)
            cy2 = jnp.full((16,), cy2s, jnp.float32)
            cx2 = jnp.full((16,), cx2s, jnp.float32)
            careas = (cy2s - cy1s) * (cx2s - cx1s)
            carea = jnp.full((16,), careas, jnp.float32)

            newsv0 = jnp.where(iota == lanew, negv, sv0)
            newsv1 = jnp.where(iota == lanew - 16, negv, sv1)
            s_v[pl.ds(cstar * 32, 16)] = newsv0
            s_v[pl.ds(cstar * 32 + 16, 16)] = newsv1
            newm = jnp.max(jnp.maximum(newsv0, newsv1))
            newmv = jnp.full((16,), newm, jnp.float32)
            kreg = cstar // 16
            klane = cstar - kreg * 16
            cms_new = tuple(
                jnp.where(kreg == k, jnp.where(iota == klane, newmv, cms[k]),
                          cms[k])
                for k in range(_NREG))

            def iou_supp(ky1, kx1, ky2, kx2, karea):
                yy1 = jnp.maximum(ky1, cy1)
                xx1 = jnp.maximum(kx1, cx1)
                yy2 = jnp.minimum(ky2, cy2)
                xx2 = jnp.minimum(kx2, cx2)
                inter = jnp.maximum(0.0, yy2 - yy1) * jnp.maximum(0.0, xx2 - xx1)
                union = karea + carea - inter
                iou = jnp.where(union > 0, inter / union, 0.0)
                return iou > _IOU_THRESHOLD

            cproper = jnp.logical_and(cy2s > cy1s, cx2s > cx1s)
            nk = jnp.where(cproper, (kcnt + 15) // 16, 0)

            def jbody(j, suppacc):
                sl = pl.ds(j * 16, 16)
                return jnp.logical_or(
                    suppacc,
                    iou_supp(sy1_v[sl], sx1_v[sl], sy2_v[sl], sx2_v[sl],
                             sarea_v[sl]))

            suppv = lax.fori_loop(0, nk, jbody, jnp.zeros((16,), jnp.bool_))
            anysupp = jnp.max(jnp.where(suppv, 1, 0).astype(jnp.int32)) > 0
            accept = jnp.logical_and(valid, jnp.logical_not(anysupp))

            obase = (cnt // 16) * 16
            olane = cnt - obase
            am = jnp.logical_and(iota == olane, jnp.full((16,), accept))
            osl = pl.ds(obase, 16)

            def blend_f(ref, vals):
                ref[osl] = jnp.where(am, jnp.full((16,), vals, jnp.float32),
                                     ref[osl])

            blend_f(ky1_v, cy1s)
            blend_f(kx1_v, cx1s)
            blend_f(ky2_v, cy2s)
            blend_f(kx2_v, cx2s)
            blend_f(osc_v, best)
            olab_v[osl] = jnp.where(am, jnp.full((16,), clabs - 1, jnp.int32),
                                    olab_v[osl])

            keep = jnp.logical_and(accept, cproper)
            kbase = (kcnt // 16) * 16
            klne = kcnt - kbase
            amk = jnp.logical_and(iota == klne, jnp.full((16,), keep))
            ksl = pl.ds(kbase, 16)

            def blend_k(ref, vals):
                ref[ksl] = jnp.where(amk, jnp.full((16,), vals, jnp.float32),
                                     ref[ksl])

            blend_k(sy1_v, cy1s)
            blend_k(sx1_v, cx1s)
            blend_k(sy2_v, cy2s)
            blend_k(sx2_v, cx2s)
            blend_k(sarea_v, careas)

            cnt2 = cnt + jnp.where(accept, 1, 0).astype(jnp.int32)
            kcnt2 = kcnt + jnp.where(keep, 1, 0).astype(jnp.int32)
            return cnt2, kcnt2, jnp.logical_not(valid), cms_new

        lax.while_loop(cond, body,
                       (jnp.int32(0), jnp.int32(0), jnp.bool_(False), cms0))

        pltpu.sync_copy(ky1_v, oy1_hbm.at[b])
        pltpu.sync_copy(kx1_v, ox1_hbm.at[b])
        pltpu.sync_copy(ky2_v, oy2_hbm.at[b])
        pltpu.sync_copy(kx2_v, ox2_hbm.at[b])
        pltpu.sync_copy(osc_v, osc_hbm.at[b])
        pltpu.sync_copy(olab_v, olab_hbm.at[b])


_sc_nms_call = functools.partial(
    pl.kernel,
    out_type=(
        jax.ShapeDtypeStruct((_B, _OUTPAD), jnp.float32),
        jax.ShapeDtypeStruct((_B, _OUTPAD), jnp.float32),
        jax.ShapeDtypeStruct((_B, _OUTPAD), jnp.float32),
        jax.ShapeDtypeStruct((_B, _OUTPAD), jnp.float32),
        jax.ShapeDtypeStruct((_B, _OUTPAD), jnp.float32),
        jax.ShapeDtypeStruct((_B, _OUTPAD), jnp.int32),
    ),
    mesh=plsc.VectorSubcoreMesh(core_axis_name="c", subcore_axis_name="s"),
    compiler_params=pltpu.CompilerParams(needs_layout_passes=False,
                                         use_tc_tiling_on_sc=False),
    scratch_types=[
        pltpu.VMEM((_NPAD,), jnp.float32),
        pltpu.VMEM((_NPAD + 16,), jnp.int32),
        pltpu.VMEM((_N * 4 + 16,), jnp.float32),
        pltpu.VMEM((_CMPAD,), jnp.float32),
        pltpu.VMEM((_OUTPAD,), jnp.float32),
        pltpu.VMEM((_OUTPAD,), jnp.float32),
        pltpu.VMEM((_OUTPAD,), jnp.float32),
        pltpu.VMEM((_OUTPAD,), jnp.float32),
        pltpu.VMEM((_OUTPAD,), jnp.float32),
        pltpu.VMEM((_OUTPAD,), jnp.int32),
        pltpu.VMEM((_OUTPAD,), jnp.float32),
        pltpu.VMEM((_OUTPAD,), jnp.float32),
        pltpu.VMEM((_OUTPAD,), jnp.float32),
        pltpu.VMEM((_OUTPAD,), jnp.float32),
        pltpu.VMEM((_OUTPAD,), jnp.float32),
    ],
)(_sc_nms)


def kernel(boxes, classification):
    cls_t = jnp.transpose(classification, (2, 0, 1))
    s_pad, lab_pad = pl.pallas_call(
        _prep_body,
        out_shape=(
            jax.ShapeDtypeStruct((_B, _N), jnp.float32),
            jax.ShapeDtypeStruct((_B, _N), jnp.int32),
        ),
    )(cls_t)
    box_flat = boxes.reshape(_B, _N * 4)
    oy1, ox1, oy2, ox2, osc, olab = _sc_nms_call(s_pad, lab_pad, box_flat)
    out_boxes = jnp.stack(
        [oy1[:, :_MAX_DET], ox1[:, :_MAX_DET],
         oy2[:, :_MAX_DET], ox2[:, :_MAX_DET]], axis=-1)
    return out_boxes, osc[:, :_MAX_DET], olab[:, :_MAX_DET]


# speedup vs baseline: 1.1847x; 1.0008x over previous
import functools

import jax
import jax.numpy as jnp
from jax import lax
from jax.experimental import pallas as pl
from jax.experimental.pallas import tpu as pltpu
from jax.experimental.pallas import tpu_sc as plsc

_SCORE_THRESHOLD = 0.05
_IOU_THRESHOLD = 0.5
_MAX_DET = 300
_NEG = float("-inf")

_B = 8
_N = 5000
_C = 80
_NPAD = 5024
_NCHUNK = _NPAD // 32
_CMPAD = 160
_OUTPAD = 304


_CBLK = 8


def _prep_body(cls_ref, s_ref, lab_ref):
    i = pl.program_id(0)
    best = cls_ref[0]
    labv = jnp.zeros((_B, _N), jnp.int32)
    for c in range(1, _CBLK):
        v = cls_ref[c]
        upd = v > best
        best = jnp.where(upd, v, best)
        labv = jnp.where(upd, c, labv)
    labv = labv + i * _CBLK

    @pl.when(i == 0)
    def _():
        s_ref[...] = best
        lab_ref[...] = labv

    @pl.when(i > 0)
    def _():
        prev = s_ref[...]
        upd = best > prev
        s_ref[...] = jnp.where(upd, best, prev)
        lab_ref[...] = jnp.where(upd, labv, lab_ref[...])

    @pl.when(i == _C // _CBLK - 1)
    def _():
        cur = s_ref[...]
        s_ref[...] = jnp.where(cur > _SCORE_THRESHOLD, cur, _NEG)


def _sc_nms(s_hbm, lab_hbm, box_hbm,
            oy1_hbm, ox1_hbm, oy2_hbm, ox2_hbm, osc_hbm, olab_hbm,
            s_v, lab_v, box_v, cm_v,
            ky1_v, kx1_v, ky2_v, kx2_v, osc_v, olab_v,
            sy1_v, sx1_v, sy2_v, sx2_v, sarea_v):
    w = lax.axis_index("s") * 2 + lax.axis_index("c")

    @pl.when(w < _B)
    def _():
        b = w
        pltpu.sync_copy(s_hbm.at[b], s_v.at[pl.ds(0, _N)])
        pltpu.sync_copy(lab_hbm.at[b], lab_v.at[pl.ds(0, _N)])
        pltpu.sync_copy(box_hbm.at[b], box_v.at[pl.ds(0, _N * 4)])

        iota = lax.iota(jnp.int32, 16)
        negv = jnp.full((16,), _NEG, jnp.float32)

        tail = s_v[pl.ds(_NPAD - 32, 16)]
        s_v[pl.ds(_NPAD - 32, 16)] = jnp.where(iota < _N - (_NPAD - 32),
                                               tail, negv)
        s_v[pl.ds(_NPAD - 16, 16)] = negv
        m1f = jnp.full((16,), -1.0, jnp.float32)
        m1i = jnp.full((16,), -1, jnp.int32)
        z16 = jnp.zeros((16,), jnp.float32)

        for k in range(_CMPAD // 16):
            cm_v[pl.ds(16 * k, 16)] = negv
        for k in range(_OUTPAD // 16):
            sl = pl.ds(16 * k, 16)
            ky1_v[sl] = m1f
            kx1_v[sl] = m1f
            ky2_v[sl] = m1f
            kx2_v[sl] = m1f
            osc_v[sl] = m1f
            olab_v[sl] = m1i
            sy1_v[sl] = m1f
            sx1_v[sl] = m1f
            sy2_v[sl] = m1f
            sx2_v[sl] = m1f
            sarea_v[sl] = z16

        def cmbody(k, _):
            m = jnp.max(jnp.maximum(s_v[pl.ds(k * 32, 16)],
                                    s_v[pl.ds(k * 32 + 16, 16)]))
            base = (k // 16) * 16
            lane = k - base
            old = cm_v[pl.ds(base, 16)]
            cm_v[pl.ds(base, 16)] = jnp.where(iota == lane,
                                              jnp.full((16,), m, jnp.float32),
                                              old)
            return 0
        lax.fori_loop(0, _NCHUNK, cmbody, 0)

        BIG = jnp.int32(10 ** 6)
        _NREG = _CMPAD // 16

        def _tree(op, vs):
            vs = list(vs)
            while len(vs) > 1:
                vs = [op(vs[i], vs[i + 1]) if i + 1 < len(vs) else vs[i]
                      for i in range(0, len(vs), 2)]
            return vs[0]

        cms0 = tuple(cm_v[pl.ds(16 * k, 16)] for k in range(_NREG))

        def cond(st):
            cnt, done = st[0], st[2]
            return jnp.logical_and(cnt < _MAX_DET, jnp.logical_not(done))

        def body(st):
            cnt, kcnt, done, cms = st[0], st[1], st[2], st[3]
            best = jnp.max(_tree(jnp.maximum, cms))
            valid = best > _NEG
            bestv = jnp.full((16,), best, jnp.float32)
            idxs = [jnp.where(cms[k] == bestv, iota + 16 * k, BIG)
                    for k in range(_NREG)]
            cstar = jnp.min(_tree(jnp.minimum, idxs))
            sv0 = s_v[pl.ds(cstar * 32, 16)]
            sv1 = s_v[pl.ds(cstar * 32 + 16, 16)]
            l0 = jnp.min(jnp.where(sv0 == bestv, iota, BIG))
            l1 = jnp.min(jnp.where(sv1 == bestv, iota + 16, BIG))
            lanew = jnp.minimum(l0, l1)
            g = cstar * 32 + lanew
            gc = jnp.minimum(g, _N - 1)

            bv = box_v[pl.ds(gc * 4, 16)]
            cy1s = bv[0]
            cx1s = bv[1]
            cy2s = bv[2]
            cx2s = bv[3]
            clabs = lab_v[pl.ds(gc, 16)][0]
            cy1 = jnp.full((16,), cy1s, jnp.float32)
            cx1 = jnp.full((16,), cx1s, jnp.float32)
            cy2 = jnp.full((16,), cy2s, jnp.float32)
            cx2 = jnp.full((16,), cx2s, jnp.float32)
            careas = (cy2s - cy1s) * (cx2s - cx1s)
            carea = jnp.full((16,), careas, jnp.float32)

            newsv0 = jnp.where(iota == lanew, negv, sv0)
            newsv1 = jnp.where(iota == lanew - 16, negv, sv1)
            s_v[pl.ds(cstar * 32, 16)] = newsv0
            s_v[pl.ds(cstar * 32 + 16, 16)] = newsv1
            newm = jnp.max(jnp.maximum(newsv0, newsv1))
            newmv = jnp.full((16,), newm, jnp.float32)
            kreg = cstar // 16
            klane = cstar - kreg * 16
            cms_new = tuple(
                jnp.where(kreg == k, jnp.where(iota == klane, newmv, cms[k]),
                          cms[k])
                for k in range(_NREG))

            def iou_supp(ky1, kx1, ky2, kx2, karea):
                yy1 = jnp.maximum(ky1, cy1)
                xx1 = jnp.maximum(kx1, cx1)
                yy2 = jnp.minimum(ky2, cy2)
                xx2 = jnp.minimum(kx2, cx2)
                inter = jnp.maximum(0.0, yy2 - yy1) * jnp.maximum(0.0, xx2 - xx1)
                union = karea + carea - inter
                iou = jnp.where(union > 0, inter / union, 0.0)
                return iou > _IOU_THRESHOLD

            cproper = jnp.logical_and(cy2s > cy1s, cx2s > cx1s)
            nk = jnp.where(cproper, (kcnt + 15) // 16, 0)

            def jbody(j, suppacc):
                sl = pl.ds(j * 16, 16)
                return jnp.logical_or(
                    suppacc,
                    iou_supp(sy1_v[sl], sx1_v[sl], sy2_v[sl], sx2_v[sl],
                             sarea_v[sl]))

            suppv = lax.fori_loop(0, nk, jbody, jnp.zeros((16,), jnp.bool_))
            anysupp = jnp.max(jnp.where(suppv, 1, 0).astype(jnp.int32)) > 0
            accept = jnp.logical_and(valid, jnp.logical_not(anysupp))

            obase = (cnt // 16) * 16
            olane = cnt - obase
            am = jnp.logical_and(iota == olane, jnp.full((16,), accept))
            osl = pl.ds(obase, 16)

            def blend_f(ref, vals):
                ref[osl] = jnp.where(am, jnp.full((16,), vals, jnp.float32),
                                     ref[osl])

            blend_f(ky1_v, cy1s)
            blend_f(kx1_v, cx1s)
            blend_f(ky2_v, cy2s)
            blend_f(kx2_v, cx2s)
            blend_f(osc_v, best)
            olab_v[osl] = jnp.where(am, jnp.full((16,), clabs - 1, jnp.int32),
                                    olab_v[osl])

            keep = jnp.logical_and(accept, cproper)
            kbase = (kcnt // 16) * 16
            klne = kcnt - kbase
            amk = jnp.logical_and(iota == klne, jnp.full((16,), keep))
            ksl = pl.ds(kbase, 16)

            def blend_k(ref, vals):
                ref[ksl] = jnp.where(amk, jnp.full((16,), vals, jnp.float32),
                                     ref[ksl])

            blend_k(sy1_v, cy1s)
            blend_k(sx1_v, cx1s)
            blend_k(sy2_v, cy2s)
            blend_k(sx2_v, cx2s)
            blend_k(sarea_v, careas)

            cnt2 = cnt + jnp.where(accept, 1, 0).astype(jnp.int32)
            kcnt2 = kcnt + jnp.where(keep, 1, 0).astype(jnp.int32)
            return cnt2, kcnt2, jnp.logical_not(valid), cms_new

        lax.while_loop(cond, body,
                       (jnp.int32(0), jnp.int32(0), jnp.bool_(False), cms0))

        pltpu.sync_copy(ky1_v, oy1_hbm.at[b])
        pltpu.sync_copy(kx1_v, ox1_hbm.at[b])
        pltpu.sync_copy(ky2_v, oy2_hbm.at[b])
        pltpu.sync_copy(kx2_v, ox2_hbm.at[b])
        pltpu.sync_copy(osc_v, osc_hbm.at[b])
        pltpu.sync_copy(olab_v, olab_hbm.at[b])


_sc_nms_call = functools.partial(
    pl.kernel,
    out_type=(
        jax.ShapeDtypeStruct((_B, _OUTPAD), jnp.float32),
        jax.ShapeDtypeStruct((_B, _OUTPAD), jnp.float32),
        jax.ShapeDtypeStruct((_B, _OUTPAD), jnp.float32),
        jax.ShapeDtypeStruct((_B, _OUTPAD), jnp.float32),
        jax.ShapeDtypeStruct((_B, _OUTPAD), jnp.float32),
        jax.ShapeDtypeStruct((_B, _OUTPAD), jnp.int32),
    ),
    mesh=plsc.VectorSubcoreMesh(core_axis_name="c", subcore_axis_name="s"),
    compiler_params=pltpu.CompilerParams(needs_layout_passes=False,
                                         use_tc_tiling_on_sc=False),
    scratch_types=[
        pltpu.VMEM((_NPAD,), jnp.float32),
        pltpu.VMEM((_NPAD + 16,), jnp.int32),
        pltpu.VMEM((_N * 4 + 16,), jnp.float32),
        pltpu.VMEM((_CMPAD,), jnp.float32),
        pltpu.VMEM((_OUTPAD,), jnp.float32),
        pltpu.VMEM((_OUTPAD,), jnp.float32),
        pltpu.VMEM((_OUTPAD,), jnp.float32),
        pltpu.VMEM((_OUTPAD,), jnp.float32),
        pltpu.VMEM((_OUTPAD,), jnp.float32),
        pltpu.VMEM((_OUTPAD,), jnp.int32),
        pltpu.VMEM((_OUTPAD,), jnp.float32),
        pltpu.VMEM((_OUTPAD,), jnp.float32),
        pltpu.VMEM((_OUTPAD,), jnp.float32),
        pltpu.VMEM((_OUTPAD,), jnp.float32),
        pltpu.VMEM((_OUTPAD,), jnp.float32),
    ],
)(_sc_nms)


def kernel(boxes, classification):
    cls_t = jnp.transpose(classification, (2, 0, 1))
    s_pad, lab_pad = pl.pallas_call(
        _prep_body,
        grid=(_C // _CBLK,),
        in_specs=[pl.BlockSpec((_CBLK, _B, _N), lambda i: (i, 0, 0))],
        out_specs=(pl.BlockSpec((_B, _N), lambda i: (0, 0)),
                   pl.BlockSpec((_B, _N), lambda i: (0, 0))),
        out_shape=(
            jax.ShapeDtypeStruct((_B, _N), jnp.float32),
            jax.ShapeDtypeStruct((_B, _N), jnp.int32),
        ),
    )(cls_t)
    box_flat = boxes.reshape(_B, _N * 4)
    oy1, ox1, oy2, ox2, osc, olab = _sc_nms_call(s_pad, lab_pad, box_flat)
    out_boxes = jnp.stack(
        [oy1[:, :_MAX_DET], ox1[:, :_MAX_DET],
         oy2[:, :_MAX_DET], ox2[:, :_MAX_DET]], axis=-1)
    return out_boxes, osc[:, :_MAX_DET], olab[:, :_MAX_DET]
